# baseline (speedup 1.0000x reference)
import functools

import jax
import jax.numpy as jnp
from jax import lax
from jax.experimental import pallas as pl
from jax.experimental.pallas import tpu as pltpu
from jax.experimental.pallas import tpu_sc as plsc

S, D, E, K = 2048, 1024, 64, 2
FF, SFF = 1408, 2816
TM = 128
NB = 96
RPAD = NB * TM
TS = 256
D2 = D // 2


def _router_body(x_ref, gw_ref, tv_ref, ti_ref, x16_ref):
    lt = lax.dot_general(gw_ref[...], x_ref[...], (((1,), (1,)), ((), ())),
                         preferred_element_type=jnp.float32)
    m = jnp.max(lt, axis=0, keepdims=True)
    p = jnp.exp(lt - m)
    probs = p / jnp.sum(p, axis=0, keepdims=True)
    rows = lax.broadcasted_iota(jnp.int32, (E, S), 0)
    v0 = jnp.max(probs, axis=0)
    i0 = jnp.min(jnp.where(probs == v0[None, :], rows, E), axis=0)
    probs2 = jnp.where(rows == i0[None, :], -1.0, probs)
    v1 = jnp.max(probs2, axis=0)
    i1 = jnp.min(jnp.where(probs2 == v1[None, :], rows, E), axis=0)
    tv_ref[0:1, :] = v0.reshape(1, S)
    tv_ref[1:2, :] = v1.reshape(1, S)
    ti_ref[0:1, :] = i0.reshape(1, S)
    ti_ref[1:2, :] = i1.reshape(1, S)
    x16_ref[...] = x_ref[...].astype(jnp.bfloat16)


def _router(x, gate_w):
    return pl.pallas_call(
        _router_body,
        out_shape=[jax.ShapeDtypeStruct((8, S), jnp.float32),
                   jax.ShapeDtypeStruct((8, S), jnp.int32),
                   jax.ShapeDtypeStruct((S, D), jnp.bfloat16)],
        in_specs=[pl.BlockSpec((S, D), lambda: (0, 0)),
                  pl.BlockSpec((E, D), lambda: (0, 0))],
        out_specs=[pl.BlockSpec((8, S), lambda: (0, 0)),
                   pl.BlockSpec((8, S), lambda: (0, 0)),
                   pl.BlockSpec((S, D), lambda: (0, 0))],
    )(x, gate_w)


def _sc_gather_rows(x, gsrc):
    info = plsc.get_sparse_core_info()
    nc, ns = info.num_cores, info.num_subcores
    nw = nc * ns
    rows_w = RPAD // nw
    ch = 64
    nch = rows_w // ch
    mesh = plsc.VectorSubcoreMesh(core_axis_name="c", subcore_axis_name="s")

    @functools.partial(
        pl.kernel, mesh=mesh,
        out_type=jax.ShapeDtypeStruct((RPAD, D2), jnp.float32),
        scratch_types=[pltpu.VMEM((ch,), jnp.int32),
                       pltpu.VMEM((ch, D2), jnp.float32),
                       pltpu.SemaphoreType.DMA])
    def gather_k(x_hbm, gsrc_hbm, xs_hbm, idx_v, buf, sem):
        wid = lax.axis_index("s") * nc + lax.axis_index("c")

        def chunk(c, carry):
            base = wid * rows_w + c * ch
            pltpu.sync_copy(gsrc_hbm.at[pl.ds(base, ch)], idx_v)
            pltpu.async_copy(x_hbm.at[idx_v], buf, sem).wait()
            pltpu.sync_copy(buf, xs_hbm.at[pl.ds(base, ch)])
            return carry

        lax.fori_loop(0, nch, chunk, 0)

    return gather_k(x, gsrc)


def _gmm_body(beid_ref, nreal_ref, xs_ref, w1_ref, w3_ref, w2_ref, scl_ref,
              ys_ref):
    b = pl.program_id(0)

    @pl.when(b < nreal_ref[0])
    def _():
        xb = xs_ref[...]
        g = lax.dot_general(xb, w1_ref[0].astype(jnp.bfloat16),
                            (((1,), (1,)), ((), ())),
                            preferred_element_type=jnp.float32)
        u = lax.dot_general(xb, w3_ref[0].astype(jnp.bfloat16),
                            (((1,), (1,)), ((), ())),
                            preferred_element_type=jnp.float32)
        h = (g * jax.nn.sigmoid(g) * u).astype(jnp.bfloat16)
        y = lax.dot_general(h, w2_ref[0].astype(jnp.bfloat16),
                            (((1,), (1,)), ((), ())),
                            preferred_element_type=jnp.float32)
        ys_ref[...] = (y * scl_ref[...]).astype(jnp.bfloat16)


def _gmm(beid, nreal, xs, w1, w3, w2, scl):
    grid_spec = pltpu.PrefetchScalarGridSpec(
        num_scalar_prefetch=2,
        grid=(NB,),
        in_specs=[
            pl.BlockSpec((TM, D), lambda b, beid, nr: (b, 0)),
            pl.BlockSpec((1, FF, D), lambda b, beid, nr: (beid[b], 0, 0)),
            pl.BlockSpec((1, FF, D), lambda b, beid, nr: (beid[b], 0, 0)),
            pl.BlockSpec((1, D, FF), lambda b, beid, nr: (beid[b], 0, 0)),
            pl.BlockSpec((TM, 1), lambda b, beid, nr: (b, 0)),
        ],
        out_specs=pl.BlockSpec((TM, D), lambda b, beid, nr: (b, 0)),
    )
    return pl.pallas_call(
        _gmm_body,
        grid_spec=grid_spec,
        out_shape=jax.ShapeDtypeStruct((RPAD, D), jnp.bfloat16),
        compiler_params=pltpu.CompilerParams(
            dimension_semantics=("arbitrary",)),
    )(beid, nreal, xs, w1, w3, w2, scl)


def _sc_gather_pairs(ys, pp):
    info = plsc.get_sparse_core_info()
    nc, ns = info.num_cores, info.num_subcores
    nw = nc * ns
    tok_w = S // nw
    ch = 32
    nch = tok_w // ch
    mesh = plsc.VectorSubcoreMesh(core_axis_name="c", subcore_axis_name="s")

    @functools.partial(
        pl.kernel, mesh=mesh,
        out_type=[jax.ShapeDtypeStruct((S, D2), jnp.float32),
                  jax.ShapeDtypeStruct((S, D2), jnp.float32)],
        scratch_types=[pltpu.VMEM((ch,), jnp.int32),
                       pltpu.VMEM((ch, D2), jnp.float32),
                       pltpu.SemaphoreType.DMA])
    def comb_k(ys_hbm, pp_hbm, y0_hbm, y1_hbm, idx_v, buf, sem):
        wid = lax.axis_index("s") * nc + lax.axis_index("c")

        def chunk(c, carry):
            tb = wid * tok_w + c * ch
            pltpu.sync_copy(pp_hbm.at[pl.ds(tb, ch)], idx_v)
            pltpu.async_copy(ys_hbm.at[idx_v], buf, sem).wait()
            pltpu.sync_copy(buf, y0_hbm.at[pl.ds(tb, ch)])
            pltpu.sync_copy(pp_hbm.at[pl.ds(S + tb, ch)], idx_v)
            pltpu.async_copy(ys_hbm.at[idx_v], buf, sem).wait()
            pltpu.sync_copy(buf, y1_hbm.at[pl.ds(tb, ch)])
            return carry

        lax.fori_loop(0, nch, chunk, 0)

    return comb_k(ys, pp)


def _shared_body(x_ref, w1_ref, w3_ref, w2_ref, gw_ref, y0_ref, y1_ref,
                 o_ref):
    xb = x_ref[...]
    xb16 = xb.astype(jnp.bfloat16)
    g = lax.dot_general(xb16, w1_ref[...].astype(jnp.bfloat16),
                        (((1,), (1,)), ((), ())),
                        preferred_element_type=jnp.float32)
    u = lax.dot_general(xb16, w3_ref[...].astype(jnp.bfloat16),
                        (((1,), (1,)), ((), ())),
                        preferred_element_type=jnp.float32)
    h = (g * jax.nn.sigmoid(g) * u).astype(jnp.bfloat16)
    sh = lax.dot_general(h, w2_ref[...].astype(jnp.bfloat16),
                         (((1,), (1,)), ((), ())),
                         preferred_element_type=jnp.float32)
    gate = jax.nn.sigmoid(jnp.sum(xb * gw_ref[...], axis=1, keepdims=True))
    o_ref[...] = (y0_ref[...].astype(jnp.float32) +
                  y1_ref[...].astype(jnp.float32) + sh * gate)


def _shared(x, w1s, w3s, w2s, sgw, y0, y1):
    return pl.pallas_call(
        _shared_body,
        grid=(S // TS,),
        in_specs=[
            pl.BlockSpec((TS, D), lambda t: (t, 0)),
            pl.BlockSpec((SFF, D), lambda t: (0, 0)),
            pl.BlockSpec((SFF, D), lambda t: (0, 0)),
            pl.BlockSpec((D, SFF), lambda t: (0, 0)),
            pl.BlockSpec((1, D), lambda t: (0, 0)),
            pl.BlockSpec((TS, D), lambda t: (t, 0)),
            pl.BlockSpec((TS, D), lambda t: (t, 0)),
        ],
        out_specs=pl.BlockSpec((TS, D), lambda t: (t, 0)),
        out_shape=jax.ShapeDtypeStruct((S, D), jnp.float32),
        compiler_params=pltpu.CompilerParams(
            dimension_semantics=("arbitrary",)),
    )(x, w1s, w3s, w2s, sgw, y0, y1)


def kernel(hidden_states, gate_w, w1, w2, w3, shared_w1, shared_w2,
           shared_w3, shared_gate_w):
    orig_shape = hidden_states.shape
    x = hidden_states.reshape(S, D)

    tv, ti, x16 = _router(x, gate_w)
    se = ti[:K].reshape(-1)
    tvv = tv[:K].reshape(-1)

    order = jnp.argsort(se)
    sorted_e = se[order]
    counts = jnp.zeros((E,), jnp.int32).at[se].add(1)
    blocks_e = (counts + TM - 1) // TM
    blk_cum = jnp.cumsum(blocks_e)
    nreal = blk_cum[-1].astype(jnp.int32)
    bstart = (blk_cum - blocks_e) * TM
    gstart = jnp.cumsum(counts) - counts
    r4 = jnp.arange(S * K, dtype=jnp.int32)
    rank = r4 - gstart[sorted_e].astype(jnp.int32)
    dest = bstart[sorted_e].astype(jnp.int32) + rank
    tok_sorted = (order % S).astype(jnp.int32)
    gsrc = (jnp.arange(RPAD, dtype=jnp.int32) % S).at[dest].set(tok_sorted)
    scl = jnp.zeros((RPAD,), jnp.float32).at[dest].set(tvv[order])
    beid_raw = jnp.searchsorted(blk_cum, jnp.arange(NB), side="right")
    beid = jnp.where(jnp.arange(NB) < nreal, beid_raw,
                     sorted_e[-1]).astype(jnp.int32)
    pp = jnp.zeros((S * K,), jnp.int32).at[order].set(dest)

    x32 = lax.bitcast_convert_type(x16.reshape(S, D2, 2), jnp.float32)
    xs32 = _sc_gather_rows(x32, gsrc)
    xs = lax.bitcast_convert_type(xs32, jnp.bfloat16).reshape(RPAD, D)
    ys = _gmm(beid, nreal.reshape(1), xs, w1, w3, w2, scl.reshape(RPAD, 1))
    ys32 = lax.bitcast_convert_type(ys.reshape(RPAD, D2, 2), jnp.float32)
    y0_32, y1_32 = _sc_gather_pairs(ys32, pp)
    y0 = lax.bitcast_convert_type(y0_32, jnp.bfloat16).reshape(S, D)
    y1 = lax.bitcast_convert_type(y1_32, jnp.bfloat16).reshape(S, D)
    out = _shared(x, shared_w1, shared_w3, shared_w2, shared_gate_w, y0, y1)
    return out.reshape(orig_shape)


# speedup vs baseline: 2.0169x; 2.0169x over previous
import functools

import jax
import jax.numpy as jnp
from jax import lax
from jax.experimental import pallas as pl
from jax.experimental.pallas import tpu as pltpu
from jax.experimental.pallas import tpu_sc as plsc

S, D, E, K = 2048, 1024, 64, 2
FF, SFF = 1408, 2816
TM = 128
NB = 96
RPAD = NB * TM
TS = 256


def _router_body(x_ref, gw_ref, tv_ref, ti_ref):
    lt = lax.dot_general(gw_ref[...], x_ref[...], (((1,), (1,)), ((), ())),
                         preferred_element_type=jnp.float32)
    m = jnp.max(lt, axis=0, keepdims=True)
    p = jnp.exp(lt - m)
    probs = p / jnp.sum(p, axis=0, keepdims=True)
    rows = lax.broadcasted_iota(jnp.int32, (E, S), 0)
    v0 = jnp.max(probs, axis=0)
    i0 = jnp.min(jnp.where(probs == v0[None, :], rows, E), axis=0)
    probs2 = jnp.where(rows == i0[None, :], -1.0, probs)
    v1 = jnp.max(probs2, axis=0)
    i1 = jnp.min(jnp.where(probs2 == v1[None, :], rows, E), axis=0)
    tv_ref[0:1, :] = v0.reshape(1, S)
    tv_ref[1:2, :] = v1.reshape(1, S)
    ti_ref[0:1, :] = i0.reshape(1, S)
    ti_ref[1:2, :] = i1.reshape(1, S)


def _router(x, gate_w):
    return pl.pallas_call(
        _router_body,
        out_shape=[jax.ShapeDtypeStruct((8, S), jnp.float32),
                   jax.ShapeDtypeStruct((8, S), jnp.int32)],
        in_specs=[pl.BlockSpec((S, D), lambda: (0, 0)),
                  pl.BlockSpec((E, D), lambda: (0, 0))],
        out_specs=[pl.BlockSpec((8, S), lambda: (0, 0)),
                   pl.BlockSpec((8, S), lambda: (0, 0))],
    )(x, gate_w)


def _sc_gather_rows(x, gsrc):
    info = plsc.get_sparse_core_info()
    nc, ns = info.num_cores, info.num_subcores
    nw = nc * ns
    rows_w = RPAD // nw
    ch = 64
    nch = rows_w // ch
    mesh = plsc.VectorSubcoreMesh(core_axis_name="c", subcore_axis_name="s")

    @functools.partial(
        pl.kernel, mesh=mesh,
        out_type=jax.ShapeDtypeStruct((RPAD, D), jnp.float32),
        scratch_types=[pltpu.VMEM((ch,), jnp.int32),
                       pltpu.VMEM((ch, D), jnp.float32),
                       pltpu.SemaphoreType.DMA])
    def gather_k(x_hbm, gsrc_hbm, xs_hbm, idx_v, buf, sem):
        wid = lax.axis_index("s") * nc + lax.axis_index("c")

        def chunk(c, carry):
            base = wid * rows_w + c * ch
            pltpu.sync_copy(gsrc_hbm.at[pl.ds(base, ch)], idx_v)
            pltpu.async_copy(x_hbm.at[idx_v], buf, sem).wait()
            pltpu.sync_copy(buf, xs_hbm.at[pl.ds(base, ch)])
            return carry

        lax.fori_loop(0, nch, chunk, 0)

    return gather_k(x, gsrc)


def _gmm_body(beid_ref, nreal_ref, xs_ref, w1_ref, w3_ref, w2_ref, scl_ref,
              ys_ref):
    b = pl.program_id(0)

    @pl.when(b < nreal_ref[0])
    def _():
        xb = xs_ref[...].astype(jnp.bfloat16)
        g = lax.dot_general(xb, w1_ref[0].astype(jnp.bfloat16),
                            (((1,), (1,)), ((), ())),
                            preferred_element_type=jnp.float32)
        u = lax.dot_general(xb, w3_ref[0].astype(jnp.bfloat16),
                            (((1,), (1,)), ((), ())),
                            preferred_element_type=jnp.float32)
        h = (g * jax.nn.sigmoid(g) * u).astype(jnp.bfloat16)
        y = lax.dot_general(h, w2_ref[0].astype(jnp.bfloat16),
                            (((1,), (1,)), ((), ())),
                            preferred_element_type=jnp.float32)
        ys_ref[...] = y * scl_ref[...]


def _gmm(beid, nreal, xs, w1, w3, w2, scl):
    grid_spec = pltpu.PrefetchScalarGridSpec(
        num_scalar_prefetch=2,
        grid=(NB,),
        in_specs=[
            pl.BlockSpec((TM, D), lambda b, beid, nr: (b, 0)),
            pl.BlockSpec((1, FF, D), lambda b, beid, nr: (beid[b], 0, 0)),
            pl.BlockSpec((1, FF, D), lambda b, beid, nr: (beid[b], 0, 0)),
            pl.BlockSpec((1, D, FF), lambda b, beid, nr: (beid[b], 0, 0)),
            pl.BlockSpec((TM, 1), lambda b, beid, nr: (b, 0)),
        ],
        out_specs=pl.BlockSpec((TM, D), lambda b, beid, nr: (b, 0)),
    )
    return pl.pallas_call(
        _gmm_body,
        grid_spec=grid_spec,
        out_shape=jax.ShapeDtypeStruct((RPAD, D), jnp.float32),
        compiler_params=pltpu.CompilerParams(
            dimension_semantics=("arbitrary",)),
    )(beid, nreal, xs, w1, w3, w2, scl)


def _sc_gather_pairs(ys, pp):
    info = plsc.get_sparse_core_info()
    nc, ns = info.num_cores, info.num_subcores
    nw = nc * ns
    tok_w = S // nw
    ch = 32
    nch = tok_w // ch
    mesh = plsc.VectorSubcoreMesh(core_axis_name="c", subcore_axis_name="s")

    @functools.partial(
        pl.kernel, mesh=mesh,
        out_type=[jax.ShapeDtypeStruct((S, D), jnp.float32),
                  jax.ShapeDtypeStruct((S, D), jnp.float32)],
        scratch_types=[pltpu.VMEM((ch,), jnp.int32),
                       pltpu.VMEM((ch, D), jnp.float32),
                       pltpu.SemaphoreType.DMA])
    def comb_k(ys_hbm, pp_hbm, y0_hbm, y1_hbm, idx_v, buf, sem):
        wid = lax.axis_index("s") * nc + lax.axis_index("c")

        def chunk(c, carry):
            tb = wid * tok_w + c * ch
            pltpu.sync_copy(pp_hbm.at[pl.ds(tb, ch)], idx_v)
            pltpu.async_copy(ys_hbm.at[idx_v], buf, sem).wait()
            pltpu.sync_copy(buf, y0_hbm.at[pl.ds(tb, ch)])
            pltpu.sync_copy(pp_hbm.at[pl.ds(S + tb, ch)], idx_v)
            pltpu.async_copy(ys_hbm.at[idx_v], buf, sem).wait()
            pltpu.sync_copy(buf, y1_hbm.at[pl.ds(tb, ch)])
            return carry

        lax.fori_loop(0, nch, chunk, 0)

    return comb_k(ys, pp)


def _shared_body(x_ref, w1_ref, w3_ref, w2_ref, gw_ref, o_ref):
    xb = x_ref[...]
    xb16 = xb.astype(jnp.bfloat16)
    g = lax.dot_general(xb16, w1_ref[...].astype(jnp.bfloat16),
                        (((1,), (1,)), ((), ())),
                        preferred_element_type=jnp.float32)
    u = lax.dot_general(xb16, w3_ref[...].astype(jnp.bfloat16),
                        (((1,), (1,)), ((), ())),
                        preferred_element_type=jnp.float32)
    h = (g * jax.nn.sigmoid(g) * u).astype(jnp.bfloat16)
    sh = lax.dot_general(h, w2_ref[...].astype(jnp.bfloat16),
                         (((1,), (1,)), ((), ())),
                         preferred_element_type=jnp.float32)
    gate = jax.nn.sigmoid(jnp.sum(xb * gw_ref[...], axis=1, keepdims=True))
    o_ref[...] = sh * gate


def _shared(x, w1s, w3s, w2s, sgw):
    return pl.pallas_call(
        _shared_body,
        grid=(S // TS,),
        in_specs=[
            pl.BlockSpec((TS, D), lambda t: (t, 0)),
            pl.BlockSpec((SFF, D), lambda t: (0, 0)),
            pl.BlockSpec((SFF, D), lambda t: (0, 0)),
            pl.BlockSpec((D, SFF), lambda t: (0, 0)),
            pl.BlockSpec((1, D), lambda t: (0, 0)),
        ],
        out_specs=pl.BlockSpec((TS, D), lambda t: (t, 0)),
        out_shape=jax.ShapeDtypeStruct((S, D), jnp.float32),
        compiler_params=pltpu.CompilerParams(
            dimension_semantics=("arbitrary",)),
    )(x, w1s, w3s, w2s, sgw)


def _combine_body(p_ref, y0_ref, y1_ref, o_ref):
    o_ref[...] = p_ref[...] + y0_ref[...] + y1_ref[...]


def _combine(partial, y0, y1):
    return pl.pallas_call(
        _combine_body,
        grid=(S // TS,),
        in_specs=[pl.BlockSpec((TS, D), lambda t: (t, 0)),
                  pl.BlockSpec((TS, D), lambda t: (t, 0)),
                  pl.BlockSpec((TS, D), lambda t: (t, 0))],
        out_specs=pl.BlockSpec((TS, D), lambda t: (t, 0)),
        out_shape=jax.ShapeDtypeStruct((S, D), jnp.float32),
        compiler_params=pltpu.CompilerParams(
            dimension_semantics=("arbitrary",)),
    )(partial, y0, y1)


def kernel(hidden_states, gate_w, w1, w2, w3, shared_w1, shared_w2,
           shared_w3, shared_gate_w):
    orig_shape = hidden_states.shape
    x = hidden_states.reshape(S, D)

    tv, ti = _router(x, gate_w)
    se = ti[:K].reshape(-1)
    tvv = tv[:K].reshape(-1)

    order = jnp.argsort(se)
    sorted_e = se[order]
    counts = jnp.zeros((E,), jnp.int32).at[se].add(1)
    blocks_e = (counts + TM - 1) // TM
    blk_cum = jnp.cumsum(blocks_e)
    nreal = blk_cum[-1].astype(jnp.int32)
    bstart = (blk_cum - blocks_e) * TM
    gstart = jnp.cumsum(counts) - counts
    r4 = jnp.arange(S * K, dtype=jnp.int32)
    rank = r4 - gstart[sorted_e].astype(jnp.int32)
    dest = bstart[sorted_e].astype(jnp.int32) + rank
    tok_sorted = (order % S).astype(jnp.int32)
    gsrc = (jnp.arange(RPAD, dtype=jnp.int32) % S).at[dest].set(tok_sorted)
    scl = jnp.zeros((RPAD,), jnp.float32).at[dest].set(tvv[order])
    beid_raw = jnp.searchsorted(blk_cum, jnp.arange(NB), side="right")
    beid = jnp.where(jnp.arange(NB) < nreal, beid_raw,
                     sorted_e[-1]).astype(jnp.int32)
    pp = jnp.zeros((S * K,), jnp.int32).at[order].set(dest)

    partial = _shared(x, shared_w1, shared_w3, shared_w2, shared_gate_w)
    xs = _sc_gather_rows(x, gsrc)
    ys = _gmm(beid, nreal.reshape(1), xs, w1, w3, w2, scl.reshape(RPAD, 1))
    y0, y1 = _sc_gather_pairs(ys, pp)
    out = _combine(partial, y0, y1)
    return out.reshape(orig_shape)


# speedup vs baseline: 2.0872x; 1.0348x over previous
import functools

import jax
import jax.numpy as jnp
from jax import lax
from jax.experimental import pallas as pl
from jax.experimental.pallas import tpu as pltpu
from jax.experimental.pallas import tpu_sc as plsc

S, D, E, K = 2048, 1024, 64, 2
FF, SFF = 1408, 2816
TM = 128
NB = 96
RPAD = NB * TM
TS = 256


def _router_body(x_ref, gw_ref, tv_ref, ti_ref):
    lt = lax.dot_general(gw_ref[...], x_ref[...], (((1,), (1,)), ((), ())),
                         preferred_element_type=jnp.float32)
    m = jnp.max(lt, axis=0, keepdims=True)
    p = jnp.exp(lt - m)
    probs = p / jnp.sum(p, axis=0, keepdims=True)
    rows = lax.broadcasted_iota(jnp.int32, (E, S), 0)
    v0 = jnp.max(probs, axis=0)
    i0 = jnp.min(jnp.where(probs == v0[None, :], rows, E), axis=0)
    probs2 = jnp.where(rows == i0[None, :], -1.0, probs)
    v1 = jnp.max(probs2, axis=0)
    i1 = jnp.min(jnp.where(probs2 == v1[None, :], rows, E), axis=0)
    tv_ref[0:1, :] = v0.reshape(1, S)
    tv_ref[1:2, :] = v1.reshape(1, S)
    ti_ref[0:1, :] = i0.reshape(1, S)
    ti_ref[1:2, :] = i1.reshape(1, S)


def _router(x, gate_w):
    return pl.pallas_call(
        _router_body,
        out_shape=[jax.ShapeDtypeStruct((8, S), jnp.float32),
                   jax.ShapeDtypeStruct((8, S), jnp.int32)],
        in_specs=[pl.BlockSpec((S, D), lambda: (0, 0)),
                  pl.BlockSpec((E, D), lambda: (0, 0))],
        out_specs=[pl.BlockSpec((8, S), lambda: (0, 0)),
                   pl.BlockSpec((8, S), lambda: (0, 0))],
    )(x, gate_w)


def _sc_gather_rows(x, gsrc):
    info = plsc.get_sparse_core_info()
    nc, ns = info.num_cores, info.num_subcores
    nw = nc * ns
    rows_w = RPAD // nw
    ch = 64
    nch = rows_w // ch
    mesh = plsc.VectorSubcoreMesh(core_axis_name="c", subcore_axis_name="s")

    @functools.partial(
        pl.kernel, mesh=mesh,
        out_type=jax.ShapeDtypeStruct((RPAD, D), jnp.float32),
        scratch_types=[pltpu.VMEM((ch,), jnp.int32),
                       pltpu.VMEM((ch, D), jnp.float32),
                       pltpu.SemaphoreType.DMA])
    def gather_k(x_hbm, gsrc_hbm, xs_hbm, idx_v, buf, sem):
        wid = lax.axis_index("s") * nc + lax.axis_index("c")

        def chunk(c, carry):
            base = wid * rows_w + c * ch
            pltpu.sync_copy(gsrc_hbm.at[pl.ds(base, ch)], idx_v)
            pltpu.async_copy(x_hbm.at[idx_v], buf, sem).wait()
            pltpu.sync_copy(buf, xs_hbm.at[pl.ds(base, ch)])
            return carry

        lax.fori_loop(0, nch, chunk, 0)

    return gather_k(x, gsrc)


def _gmm_body(beid_ref, nreal_ref, xs_ref, w1_ref, w3_ref, w2_ref, scl_ref,
              ys_ref):
    b = pl.program_id(0)

    @pl.when(b < nreal_ref[0])
    def _():
        xb = xs_ref[...].astype(jnp.bfloat16)
        g = lax.dot_general(xb, w1_ref[0].astype(jnp.bfloat16),
                            (((1,), (1,)), ((), ())),
                            preferred_element_type=jnp.float32)
        u = lax.dot_general(xb, w3_ref[0].astype(jnp.bfloat16),
                            (((1,), (1,)), ((), ())),
                            preferred_element_type=jnp.float32)
        h = (g * jax.nn.sigmoid(g) * u).astype(jnp.bfloat16)
        y = lax.dot_general(h, w2_ref[0].astype(jnp.bfloat16),
                            (((1,), (1,)), ((), ())),
                            preferred_element_type=jnp.float32)
        ys_ref[...] = y * scl_ref[...]


def _gmm(beid, nreal, xs, w1, w3, w2, scl):
    grid_spec = pltpu.PrefetchScalarGridSpec(
        num_scalar_prefetch=2,
        grid=(NB,),
        in_specs=[
            pl.BlockSpec((TM, D),
                         lambda b, beid, nr: (jnp.minimum(b, nr[0] - 1), 0)),
            pl.BlockSpec((1, FF, D), lambda b, beid, nr: (beid[b], 0, 0)),
            pl.BlockSpec((1, FF, D), lambda b, beid, nr: (beid[b], 0, 0)),
            pl.BlockSpec((1, D, FF), lambda b, beid, nr: (beid[b], 0, 0)),
            pl.BlockSpec((TM, 1),
                         lambda b, beid, nr: (jnp.minimum(b, nr[0] - 1), 0)),
        ],
        out_specs=pl.BlockSpec(
            (TM, D), lambda b, beid, nr: (jnp.minimum(b, nr[0] - 1), 0)),
    )
    return pl.pallas_call(
        _gmm_body,
        grid_spec=grid_spec,
        out_shape=jax.ShapeDtypeStruct((RPAD, D), jnp.float32),
        compiler_params=pltpu.CompilerParams(
            dimension_semantics=("arbitrary",),
            vmem_limit_bytes=112 * 1024 * 1024),
    )(beid, nreal, xs, w1, w3, w2, scl)


def _sc_gather_pairs(ys, pp):
    info = plsc.get_sparse_core_info()
    nc, ns = info.num_cores, info.num_subcores
    nw = nc * ns
    tok_w = S // nw
    ch = 32
    nch = tok_w // ch
    mesh = plsc.VectorSubcoreMesh(core_axis_name="c", subcore_axis_name="s")

    @functools.partial(
        pl.kernel, mesh=mesh,
        out_type=[jax.ShapeDtypeStruct((S, D), jnp.float32),
                  jax.ShapeDtypeStruct((S, D), jnp.float32)],
        scratch_types=[pltpu.VMEM((ch,), jnp.int32),
                       pltpu.VMEM((ch, D), jnp.float32),
                       pltpu.SemaphoreType.DMA])
    def comb_k(ys_hbm, pp_hbm, y0_hbm, y1_hbm, idx_v, buf, sem):
        wid = lax.axis_index("s") * nc + lax.axis_index("c")

        def chunk(c, carry):
            tb = wid * tok_w + c * ch
            pltpu.sync_copy(pp_hbm.at[pl.ds(tb, ch)], idx_v)
            pltpu.async_copy(ys_hbm.at[idx_v], buf, sem).wait()
            pltpu.sync_copy(buf, y0_hbm.at[pl.ds(tb, ch)])
            pltpu.sync_copy(pp_hbm.at[pl.ds(S + tb, ch)], idx_v)
            pltpu.async_copy(ys_hbm.at[idx_v], buf, sem).wait()
            pltpu.sync_copy(buf, y1_hbm.at[pl.ds(tb, ch)])
            return carry

        lax.fori_loop(0, nch, chunk, 0)

    return comb_k(ys, pp)


def _shared_body(x_ref, w1_ref, w3_ref, w2_ref, gw_ref, o_ref):
    xb = x_ref[...]
    xb16 = xb.astype(jnp.bfloat16)
    g = lax.dot_general(xb16, w1_ref[...].astype(jnp.bfloat16),
                        (((1,), (1,)), ((), ())),
                        preferred_element_type=jnp.float32)
    u = lax.dot_general(xb16, w3_ref[...].astype(jnp.bfloat16),
                        (((1,), (1,)), ((), ())),
                        preferred_element_type=jnp.float32)
    h = (g * jax.nn.sigmoid(g) * u).astype(jnp.bfloat16)
    sh = lax.dot_general(h, w2_ref[...].astype(jnp.bfloat16),
                         (((1,), (1,)), ((), ())),
                         preferred_element_type=jnp.float32)
    gate = jax.nn.sigmoid(jnp.sum(xb * gw_ref[...], axis=1, keepdims=True))
    o_ref[...] = sh * gate


def _shared(x, w1s, w3s, w2s, sgw):
    return pl.pallas_call(
        _shared_body,
        grid=(S // TS,),
        in_specs=[
            pl.BlockSpec((TS, D), lambda t: (t, 0)),
            pl.BlockSpec((SFF, D), lambda t: (0, 0)),
            pl.BlockSpec((SFF, D), lambda t: (0, 0)),
            pl.BlockSpec((D, SFF), lambda t: (0, 0)),
            pl.BlockSpec((1, D), lambda t: (0, 0)),
        ],
        out_specs=pl.BlockSpec((TS, D), lambda t: (t, 0)),
        out_shape=jax.ShapeDtypeStruct((S, D), jnp.float32),
        compiler_params=pltpu.CompilerParams(
            dimension_semantics=("arbitrary",)),
    )(x, w1s, w3s, w2s, sgw)


def _combine_body(p_ref, y0_ref, y1_ref, o_ref):
    o_ref[...] = p_ref[...] + y0_ref[...] + y1_ref[...]


def _combine(partial, y0, y1):
    return pl.pallas_call(
        _combine_body,
        grid=(S // TS,),
        in_specs=[pl.BlockSpec((TS, D), lambda t: (t, 0)),
                  pl.BlockSpec((TS, D), lambda t: (t, 0)),
                  pl.BlockSpec((TS, D), lambda t: (t, 0))],
        out_specs=pl.BlockSpec((TS, D), lambda t: (t, 0)),
        out_shape=jax.ShapeDtypeStruct((S, D), jnp.float32),
        compiler_params=pltpu.CompilerParams(
            dimension_semantics=("arbitrary",)),
    )(partial, y0, y1)


def kernel(hidden_states, gate_w, w1, w2, w3, shared_w1, shared_w2,
           shared_w3, shared_gate_w):
    orig_shape = hidden_states.shape
    x = hidden_states.reshape(S, D)

    tv, ti = _router(x, gate_w)
    se = ti[:K].reshape(-1)
    tvv = tv[:K].reshape(-1)

    order = jnp.argsort(se)
    sorted_e = se[order]
    counts = jnp.zeros((E,), jnp.int32).at[se].add(1)
    blocks_e = (counts + TM - 1) // TM
    blk_cum = jnp.cumsum(blocks_e)
    nreal = blk_cum[-1].astype(jnp.int32)
    bstart = (blk_cum - blocks_e) * TM
    gstart = jnp.cumsum(counts) - counts
    r4 = jnp.arange(S * K, dtype=jnp.int32)
    rank = r4 - gstart[sorted_e].astype(jnp.int32)
    dest = bstart[sorted_e].astype(jnp.int32) + rank
    tok_sorted = (order % S).astype(jnp.int32)
    gsrc = (jnp.arange(RPAD, dtype=jnp.int32) % S).at[dest].set(tok_sorted)
    scl = jnp.zeros((RPAD,), jnp.float32).at[dest].set(tvv[order])
    beid_raw = jnp.searchsorted(blk_cum, jnp.arange(NB), side="right")
    beid = jnp.where(jnp.arange(NB) < nreal, beid_raw,
                     sorted_e[-1]).astype(jnp.int32)
    pp = jnp.zeros((S * K,), jnp.int32).at[order].set(dest)

    partial = _shared(x, shared_w1, shared_w3, shared_w2, shared_gate_w)
    xs = _sc_gather_rows(x, gsrc)
    ys = _gmm(beid, nreal.reshape(1), xs, w1, w3, w2, scl.reshape(RPAD, 1))
    y0, y1 = _sc_gather_pairs(ys, pp)
    out = _combine(partial, y0, y1)
    return out.reshape(orig_shape)
